# baseline (speedup 1.0000x reference)
import functools

import jax
import jax.numpy as jnp
from jax import lax
from jax.experimental import pallas as pl
from jax.experimental.pallas import tpu as pltpu
from jax.experimental.pallas import tpu_sc as plsc

_HID = 1024
_NH = _HID // 16
_N_ROWS = 4096
_DEPTH = 3


def _octree_body(lvl_hbm, lemb, yemb, zemb, xemb, out_hbm,
                 lvl_v, lrow, yrow, zbase, xtab, ob0, ob1, ob2,
                 sem_g, sem_t, sem_x, sem0, sem1, sem2):
    c = lax.axis_index("c")
    s = lax.axis_index("s")
    y = s
    z0 = 9 * c

    pltpu.sync_copy(lvl_hbm, lvl_v)
    cp_l = pltpu.async_copy(lemb.at[lvl_v], lrow, sem_g)
    cp_y = pltpu.async_copy(yemb.at[pl.ds(y, 1)], yrow, sem_t)
    cp_z = pltpu.async_copy(zemb, zbase, sem_t)
    cp_x = pltpu.async_copy(xemb, xtab, sem_x)
    cp_l.wait()
    cp_y.wait()
    cp_z.wait()

    def fold(h, carry):
        hs = pl.ds(h * 16, 16)
        b = lrow[0, hs] + yrow[0, hs]
        for z in range(16):
            zbase[z, hs] = zbase[z, hs] + b
        return carry

    lax.fori_loop(0, _NH, fold, 0)
    cp_x.wait()

    obufs = (ob0, ob1, ob2)
    sems = (sem0, sem1, sem2)
    row_base = y * 256 + z0 * 16

    def emit(ng, zoff):
        pending = [None] * _DEPTH
        for g in range(ng):
            slot = g % _DEPTH
            buf = obufs[slot]
            if pending[slot] is not None:
                pending[slot].wait()

            @plsc.parallel_loop(0, _NH, 1, unroll=2)
            def hbody(h, _z=zoff + g, _buf=buf):
                hs = pl.ds(h * 16, 16)
                bv = zbase[_z, hs]
                for x in range(16):
                    _buf[x, hs] = bv + xtab[x, hs]

            pending[slot] = pltpu.async_copy(
                buf, out_hbm.at[pl.ds(row_base + g * 16, 16)], sems[slot])
        for p in pending:
            if p is not None:
                p.wait()

    @pl.when(c == 0)
    def _():
        emit(9, 0)

    @pl.when(c == 1)
    def _():
        emit(7, 9)


_mesh = plsc.VectorSubcoreMesh(core_axis_name="c", subcore_axis_name="s")

_octree = functools.partial(
    pl.kernel,
    mesh=_mesh,
    out_type=jax.ShapeDtypeStruct((_N_ROWS, _HID), jnp.float32),
    scratch_types=[
        pltpu.VMEM((1,), jnp.int32),
        pltpu.VMEM((1, _HID), jnp.float32),
        pltpu.VMEM((1, _HID), jnp.float32),
        pltpu.VMEM((16, _HID), jnp.float32),
        pltpu.VMEM((16, _HID), jnp.float32),
        pltpu.VMEM((16, _HID), jnp.float32),
        pltpu.VMEM((16, _HID), jnp.float32),
        pltpu.VMEM((16, _HID), jnp.float32),
        pltpu.SemaphoreType.DMA,
        pltpu.SemaphoreType.DMA,
        pltpu.SemaphoreType.DMA,
        pltpu.SemaphoreType.DMA,
        pltpu.SemaphoreType.DMA,
        pltpu.SemaphoreType.DMA,
    ],
)(_octree_body)


def kernel(level, level_emb, y_emb, z_emb, x_emb):
    lvl = jnp.asarray(level, jnp.int32).reshape((1,))
    return _octree(lvl, level_emb, y_emb, z_emb, x_emb)


# speedup vs baseline: 1.1696x; 1.1696x over previous
import functools

import jax
import jax.numpy as jnp
from jax import lax
from jax.experimental import pallas as pl
from jax.experimental.pallas import tpu as pltpu
from jax.experimental.pallas import tpu_sc as plsc

_HID = 1024
_NH = _HID // 16
_N_ROWS = 4096


def _octree_body(lvl_hbm, lemb, yemb, zemb, xemb, out_hbm,
                 lvl_v, lrow, yrow, zbase, xtab, ob0, ob1,
                 sem_g, sem_t, sem_x, sem0, sem1):
    c = lax.axis_index("c")
    s = lax.axis_index("s")
    w = s * 2 + c
    y = s
    half = c

    pltpu.sync_copy(lvl_hbm, lvl_v)
    cp_l = pltpu.async_copy(lemb.at[lvl_v], lrow, sem_g)
    cp_y = pltpu.async_copy(yemb.at[pl.ds(y, 1)], yrow, sem_t)
    cp_z = pltpu.async_copy(zemb.at[pl.ds(half * 8, 8)], zbase, sem_t)
    cp_x = pltpu.async_copy(xemb, xtab, sem_x)
    cp_l.wait()
    cp_y.wait()
    cp_z.wait()

    def fold(h, carry):
        hs = pl.ds(h * 16, 16)
        b = lrow[0, hs] + yrow[0, hs]
        for z in range(8):
            zbase[z, hs] = zbase[z, hs] + b
        return carry

    lax.fori_loop(0, _NH, fold, 0)
    cp_x.wait()

    obufs = (ob0, ob1)
    sems = (sem0, sem1)
    pending = [None, None]
    row0 = w * 128
    for g in range(4):
        buf = obufs[g % 2]
        if pending[g % 2] is not None:
            pending[g % 2].wait()

        @plsc.parallel_loop(0, _NH, 1, unroll=2)
        def hbody(h, _g=g, _buf=buf):
            hs = pl.ds(h * 16, 16)
            b0 = zbase[2 * _g, hs]
            b1 = zbase[2 * _g + 1, hs]
            for x in range(16):
                xv = xtab[x, hs]
                _buf[x, hs] = b0 + xv
                _buf[16 + x, hs] = b1 + xv

        pending[g % 2] = pltpu.async_copy(
            buf, out_hbm.at[pl.ds(row0 + g * 32, 32)], sems[g % 2])
    pending[0].wait()
    pending[1].wait()


_mesh = plsc.VectorSubcoreMesh(core_axis_name="c", subcore_axis_name="s")

_octree = functools.partial(
    pl.kernel,
    mesh=_mesh,
    out_type=jax.ShapeDtypeStruct((_N_ROWS, _HID), jnp.float32),
    scratch_types=[
        pltpu.VMEM((1,), jnp.int32),
        pltpu.VMEM((1, _HID), jnp.float32),
        pltpu.VMEM((1, _HID), jnp.float32),
        pltpu.VMEM((8, _HID), jnp.float32),
        pltpu.VMEM((16, _HID), jnp.float32),
        pltpu.VMEM((32, _HID), jnp.float32),
        pltpu.VMEM((32, _HID), jnp.float32),
        pltpu.SemaphoreType.DMA,
        pltpu.SemaphoreType.DMA,
        pltpu.SemaphoreType.DMA,
        pltpu.SemaphoreType.DMA,
        pltpu.SemaphoreType.DMA,
    ],
)(_octree_body)


def kernel(level, level_emb, y_emb, z_emb, x_emb):
    lvl = jnp.asarray(level, jnp.int32).reshape((1,))
    return _octree(lvl, level_emb, y_emb, z_emb, x_emb)


# speedup vs baseline: 1.2344x; 1.0553x over previous
import functools

import jax
import jax.numpy as jnp
from jax import lax
from jax.experimental import pallas as pl
from jax.experimental.pallas import tpu as pltpu
from jax.experimental.pallas import tpu_sc as plsc

_HID = 1024
_NH = _HID // 16
_N_ROWS = 4096


def _octree_body(lvl_hbm, lemb, yemb, zemb, xemb, out_hbm,
                 lvl_v, lrow, yrow, zbase8, zbase, xtab, ob0, ob1,
                 sem_g, sem_t, sem_x, sem0, sem1):
    c = lax.axis_index("c")
    s = lax.axis_index("s")
    w = s * 2 + c
    y = s
    half = c

    pltpu.sync_copy(lvl_hbm, lvl_v)
    cp_l = pltpu.async_copy(lemb.at[lvl_v], lrow, sem_g)
    cp_y = pltpu.async_copy(yemb.at[pl.ds(y, 1)], yrow, sem_t)
    cp_z = pltpu.async_copy(zemb.at[pl.ds(half * 8, 8)], zbase8, sem_t)
    cp_x = pltpu.async_copy(xemb, xtab, sem_x)
    cp_l.wait()
    cp_y.wait()
    cp_z.wait()

    def fold(h, carry):
        hs = pl.ds(h * 16, 16)
        b = lrow[0, hs] + yrow[0, hs]
        for z in range(8):
            cs = pl.ds(z * 1024 + h * 16, 16)
            zbase[0, cs] = zbase8[z, hs] + b
        return carry

    lax.fori_loop(0, _NH, fold, 0)
    cp_x.wait()

    obufs = (ob0, ob1)
    sems = (sem0, sem1)
    row0 = w * 128

    def pair(i, carry):
        for k in range(2):
            g = 2 * i + k
            buf = obufs[k]
            sem = sems[k]

            @pl.when(i > 0)
            def _():
                pltpu.make_async_copy(
                    buf, out_hbm.at[pl.ds(row0, 32)], sem).wait()

            @plsc.parallel_loop(0, _NH, 1, unroll=2)
            def hbody(h, _k=k, _buf=buf, _g=g):
                hs = pl.ds(h * 16, 16)
                b0 = zbase[0, pl.ds(2 * _g * 1024 + h * 16, 16)]
                b1 = zbase[0, pl.ds((2 * _g + 1) * 1024 + h * 16, 16)]
                for x in range(16):
                    xv = xtab[x, hs]
                    _buf[x, hs] = b0 + xv
                    _buf[16 + x, hs] = b1 + xv

            pltpu.async_copy(
                buf, out_hbm.at[pl.ds(row0 + g * 32, 32)], sem)
        return carry

    lax.fori_loop(0, 2, pair, 0)
    for k in range(2):
        pltpu.make_async_copy(
            obufs[k], out_hbm.at[pl.ds(row0, 32)], sems[k]).wait()


_mesh = plsc.VectorSubcoreMesh(core_axis_name="c", subcore_axis_name="s")

_octree = functools.partial(
    pl.kernel,
    mesh=_mesh,
    out_type=jax.ShapeDtypeStruct((_N_ROWS, _HID), jnp.float32),
    scratch_types=[
        pltpu.VMEM((1,), jnp.int32),
        pltpu.VMEM((1, _HID), jnp.float32),
        pltpu.VMEM((1, _HID), jnp.float32),
        pltpu.VMEM((8, _HID), jnp.float32),
        pltpu.VMEM((1, 8 * _HID), jnp.float32),
        pltpu.VMEM((16, _HID), jnp.float32),
        pltpu.VMEM((32, _HID), jnp.float32),
        pltpu.VMEM((32, _HID), jnp.float32),
        pltpu.SemaphoreType.DMA,
        pltpu.SemaphoreType.DMA,
        pltpu.SemaphoreType.DMA,
        pltpu.SemaphoreType.DMA,
        pltpu.SemaphoreType.DMA,
    ],
)(_octree_body)


def kernel(level, level_emb, y_emb, z_emb, x_emb):
    lvl = jnp.asarray(level, jnp.int32).reshape((1,))
    return _octree(lvl, level_emb, y_emb, z_emb, x_emb)


# speedup vs baseline: 1.3256x; 1.0739x over previous
import functools

import jax
import jax.numpy as jnp
from jax import lax
from jax.experimental import pallas as pl
from jax.experimental.pallas import tpu as pltpu
from jax.experimental.pallas import tpu_sc as plsc

_HID = 1024
_NH = _HID // 16
_N_ROWS = 4096


def _octree_body(lvl_hbm, lemb, yemb, zemb, xemb, out_hbm,
                 lvl_v, lrow, yrow, zbase8, zbase, xtab, ob0, ob1,
                 sem_g, sem_t, sem_x, sem0, sem1):
    c = lax.axis_index("c")
    s = lax.axis_index("s")
    w = s * 2 + c
    y = s
    half = c

    cp_y = pltpu.async_copy(yemb.at[pl.ds(y, 1)], yrow, sem_t)
    cp_z = pltpu.async_copy(zemb.at[pl.ds(half * 8, 8)], zbase8, sem_t)
    cp_x = pltpu.async_copy(xemb, xtab, sem_x)
    pltpu.sync_copy(lvl_hbm, lvl_v)
    cp_l = pltpu.async_copy(lemb.at[lvl_v], lrow, sem_g)
    cp_l.wait()
    cp_y.wait()
    cp_z.wait()

    def fold(h, carry):
        hs = pl.ds(h * 16, 16)
        b = lrow[0, hs] + yrow[0, hs]
        for z in range(8):
            cs = pl.ds(z * 1024 + h * 16, 16)
            zbase[0, cs] = zbase8[z, hs] + b
        return carry

    lax.fori_loop(0, _NH, fold, 0)
    cp_x.wait()

    obufs = (ob0, ob1)
    sems = (sem0, sem1)
    row0 = w * 128

    def pair(i, carry):
        for k in range(2):
            g = 2 * i + k
            buf = obufs[k]
            sem = sems[k]

            @pl.when(i > 0)
            def _():
                pltpu.make_async_copy(
                    buf, out_hbm.at[pl.ds(row0, 16)], sem).wait()

            @plsc.parallel_loop(0, _NH, 1, unroll=1)
            def hbody(h, _k=k, _buf=buf, _g=g):
                hs = pl.ds(h * 16, 16)
                b0 = zbase[0, pl.ds(_g * 1024 + h * 16, 16)]
                for x in range(16):
                    _buf[x, hs] = b0 + xtab[x, hs]

            pltpu.async_copy(
                buf, out_hbm.at[pl.ds(row0 + g * 16, 16)], sem)
        return carry

    lax.fori_loop(0, 4, pair, 0)
    for k in range(2):
        pltpu.make_async_copy(
            obufs[k], out_hbm.at[pl.ds(row0, 16)], sems[k]).wait()


_mesh = plsc.VectorSubcoreMesh(core_axis_name="c", subcore_axis_name="s")

_octree = functools.partial(
    pl.kernel,
    mesh=_mesh,
    out_type=jax.ShapeDtypeStruct((_N_ROWS, _HID), jnp.float32),
    scratch_types=[
        pltpu.VMEM((1,), jnp.int32),
        pltpu.VMEM((1, _HID), jnp.float32),
        pltpu.VMEM((1, _HID), jnp.float32),
        pltpu.VMEM((8, _HID), jnp.float32),
        pltpu.VMEM((1, 8 * _HID), jnp.float32),
        pltpu.VMEM((16, _HID), jnp.float32),
        pltpu.VMEM((16, _HID), jnp.float32),
        pltpu.VMEM((16, _HID), jnp.float32),
        pltpu.SemaphoreType.DMA,
        pltpu.SemaphoreType.DMA,
        pltpu.SemaphoreType.DMA,
        pltpu.SemaphoreType.DMA,
        pltpu.SemaphoreType.DMA,
    ],
)(_octree_body)


def kernel(level, level_emb, y_emb, z_emb, x_emb):
    lvl = jnp.asarray(level, jnp.int32).reshape((1,))
    return _octree(lvl, level_emb, y_emb, z_emb, x_emb)
